# baseline (speedup 1.0000x reference)
import functools

import jax
import jax.numpy as jnp
from jax import lax
from jax.experimental import pallas as pl
from jax.experimental.pallas import tpu as pltpu
from jax.experimental.pallas import tpu_sc as plsc

N = 10000
E = 160000
D = 128
DA = 16
DS = 4
DE = 16
H = 64
INV_SQRT_AVG = 0.25

NC = 2
NS = 16
NW = NC * NS
EPW = E // NW
CH = 40
NCH = EPW // CH
NPAD = 10112
NPS = NPAD // NS

NB_NODE = 1000
EB_EDGE = 1280
NB_OUT = 2000


def _silu(x):
    return x * (1.0 / (1.0 + jnp.exp(-x)))


def _dg0(a, b):
    return lax.dot_general(a, b, (((0,), (0,)), ((), ())),
                           preferred_element_type=jnp.float32)


def _node_body(nf_ref, na_ref, wsc_ref, w1_ref, sc_ref, nfo_ref):
    nf = nf_ref[...]
    na = na_ref[...]
    acc = na[:, 0:1] * jnp.dot(nf, wsc_ref[0], preferred_element_type=jnp.float32)
    for a in range(1, DA):
        acc = acc + na[:, a:a + 1] * jnp.dot(
            nf, wsc_ref[a], preferred_element_type=jnp.float32)
    sc_ref[...] = acc
    nfo_ref[...] = jnp.dot(nf, w1_ref[...], preferred_element_type=jnp.float32)


def _node_call(node_feats, node_attrs, wsc_r, w1):
    grid = (N // NB_NODE,)
    return pl.pallas_call(
        _node_body,
        grid=grid,
        in_specs=[
            pl.BlockSpec((NB_NODE, D), lambda i: (i, 0)),
            pl.BlockSpec((NB_NODE, DA), lambda i: (i, 0)),
            pl.BlockSpec((DA, D, D), lambda i: (0, 0, 0)),
            pl.BlockSpec((D, D), lambda i: (0, 0)),
        ],
        out_specs=[
            pl.BlockSpec((NB_NODE, D), lambda i: (i, 0)),
            pl.BlockSpec((NB_NODE, D), lambda i: (i, 0)),
        ],
        out_shape=[
            jax.ShapeDtypeStruct((N, D), jnp.float32),
            jax.ShapeDtypeStruct((N, D), jnp.float32),
        ],
    )(node_feats, node_attrs, wsc_r, w1)


def _edge_body(eft_ref, eat_ref, w0_ref, w1_ref, w2_ref, w3f_ref, g_ref):
    h = _silu(_dg0(w0_ref[...], eft_ref[...]))
    h = _silu(_dg0(w1_ref[...], h))
    h = _silu(_dg0(w2_ref[...], h))
    eat = eat_ref[...]
    hk = jnp.concatenate([eat[s:s + 1, :] * h for s in range(DS)], axis=0)
    g_ref[...] = _dg0(hk, w3f_ref[...])


def _edge_call(eft, eat, w0, w1, w2, w3f):
    grid = (E // EB_EDGE,)
    return pl.pallas_call(
        _edge_body,
        grid=grid,
        in_specs=[
            pl.BlockSpec((DE, EB_EDGE), lambda i: (0, i)),
            pl.BlockSpec((DS, EB_EDGE), lambda i: (0, i)),
            pl.BlockSpec((DE, H), lambda i: (0, 0)),
            pl.BlockSpec((H, H), lambda i: (0, 0)),
            pl.BlockSpec((H, H), lambda i: (0, 0)),
            pl.BlockSpec((DS * H, D), lambda i: (0, 0)),
        ],
        out_specs=pl.BlockSpec((EB_EDGE, D), lambda i: (i, 0)),
        out_shape=jax.ShapeDtypeStruct((E, D), jnp.float32),
    )(eft, eat, w0, w1, w2, w3f)


def _sc_scatter_body(nf_hbm, g_hbm, snd_hbm, rcv_hbm, zero_hbm, out_hbm,
                     sidx_all, ridx_all, rows0, grows0, rows1, grows1,
                     acc, sg0, sl0, sg1, sl1):
    c = lax.axis_index("c")
    s = lax.axis_index("s")
    wid = s * NC + c
    base = wid * EPW

    pltpu.sync_copy(zero_hbm, acc.at[pl.ds(s * NPS, NPS)])
    pltpu.sync_copy(snd_hbm.at[pl.ds(wid * EPW, EPW)], sidx_all)
    pltpu.sync_copy(rcv_hbm.at[wid], ridx_all)
    plsc.subcore_barrier()

    def issue(j, rows, grows, sg, sl):
        pltpu.async_copy(nf_hbm.at[sidx_all.at[pl.ds(j * CH, CH)]], rows, sg)
        pltpu.async_copy(g_hbm.at[pl.ds(base + j * CH, CH)], grows, sl)

    def wait(j, rows, grows, sg, sl):
        pltpu.make_async_copy(nf_hbm.at[sidx_all.at[pl.ds(j * CH, CH)]], rows, sg).wait()
        pltpu.make_async_copy(g_hbm.at[pl.ds(base + j * CH, CH)], grows, sl).wait()

    def mul(rows, grows):
        def mul8(i, _):
            for di in range(8):
                r = i * 8 + di
                for jj in range(D // 16):
                    sl_ = pl.ds(jj * 16, 16)
                    rows[r, sl_] = rows[r, sl_] * grows[r, sl_]
            return 0
        lax.fori_loop(0, CH // 8, mul8, 0)

    issue(0, rows0, grows0, sg0, sl0)
    issue(1, rows1, grows1, sg1, sl1)

    def pair(t, _):
        a = 2 * t
        wait(a, rows0, grows0, sg0, sl0)
        mul(rows0, grows0)
        pltpu.sync_copy(rows0, acc.at[ridx_all.at[a]], add=True)
        issue(a + 2, rows0, grows0, sg0, sl0)

        b = a + 1
        wait(b, rows1, grows1, sg1, sl1)
        mul(rows1, grows1)
        pltpu.sync_copy(rows1, acc.at[ridx_all.at[b]], add=True)

        @pl.when(t < (NCH - 1) // 2 - 1)
        def _():
            issue(b + 2, rows1, grows1, sg1, sl1)

        return 0

    lax.fori_loop(0, (NCH - 1) // 2, pair, 0)

    j = NCH - 1
    wait(j, rows0, grows0, sg0, sl0)
    mul(rows0, grows0)
    pltpu.sync_copy(rows0, acc.at[ridx_all.at[j]], add=True)

    plsc.subcore_barrier()
    pltpu.sync_copy(acc.at[pl.ds(s * NPS, NPS)],
                    out_hbm.at[pl.ds(c * NPAD + s * NPS, NPS)])


@functools.lru_cache(maxsize=1)
def _get_sc_scatter():
    mesh = plsc.VectorSubcoreMesh(core_axis_name="c", subcore_axis_name="s")
    return pl.kernel(
        _sc_scatter_body,
        mesh=mesh,
        out_type=jax.ShapeDtypeStruct((NC * NPAD, D), jnp.float32),
        scratch_types=[
            pltpu.VMEM((EPW,), jnp.int32),
            pltpu.VMEM((NCH, CH), jnp.int32),
            pltpu.VMEM((CH, D), jnp.float32),
            pltpu.VMEM((CH, D), jnp.float32),
            pltpu.VMEM((CH, D), jnp.float32),
            pltpu.VMEM((CH, D), jnp.float32),
            pltpu.VMEM_SHARED((NPAD, D), jnp.float32),
            pltpu.SemaphoreType.DMA,
            pltpu.SemaphoreType.DMA,
            pltpu.SemaphoreType.DMA,
            pltpu.SemaphoreType.DMA,
        ],
    )


def _out_body(p0_ref, p1_ref, w_ref, o_ref):
    o_ref[...] = jnp.dot(p0_ref[0] + p1_ref[0], w_ref[...],
                         preferred_element_type=jnp.float32)


def _out_call(partials, w_out):
    grid = (N // NB_OUT,)
    return pl.pallas_call(
        _out_body,
        grid=grid,
        in_specs=[
            pl.BlockSpec((1, NB_OUT, D), lambda i: (0, i, 0)),
            pl.BlockSpec((1, NB_OUT, D), lambda i: (1, i, 0)),
            pl.BlockSpec((D, D), lambda i: (0, 0)),
        ],
        out_specs=pl.BlockSpec((NB_OUT, D), lambda i: (i, 0)),
        out_shape=jax.ShapeDtypeStruct((N, D), jnp.float32),
    )(partials, partials, w_out)


def kernel(node_attrs, node_feats, edge_attrs, edge_feats, senders, receivers,
           W_sc, W1, W_m0, W_m1, W_m2, W_m3, W_out):
    wsc_r = W_sc.reshape(D, DA, D).transpose(1, 0, 2)
    w3f = (W_m3.reshape(H, D, DS).transpose(2, 0, 1).reshape(DS * H, D)
           * INV_SQRT_AVG)
    rcv2 = receivers.reshape(NW, NCH, CH)
    zeros = jnp.zeros((NPS, D), jnp.float32)

    sc, nf = _node_call(node_feats, node_attrs, wsc_r, W1)
    g = _edge_call(edge_feats.T, edge_attrs.T, W_m0, W_m1, W_m2, w3f)
    partials = _get_sc_scatter()(nf, g, senders, rcv2, zeros)
    message = _out_call(partials.reshape(NC, NPAD, D), W_out)
    return (message, sc)


# speedup vs baseline: 1.2637x; 1.2637x over previous
import functools

import jax
import jax.numpy as jnp
from jax import lax
from jax.experimental import pallas as pl
from jax.experimental.pallas import tpu as pltpu
from jax.experimental.pallas import tpu_sc as plsc

N = 10000
E = 160000
D = 128
DA = 16
DS = 4
DE = 16
H = 64
INV_SQRT_AVG = 0.25

NC = 2
NS = 16
NW = NC * NS
EPW = E // NW
CH = 40
NCH = EPW // CH
NPAD = 10112
NPS = NPAD // NS

NB_NODE = 1000
EB_EDGE = 6400
NB_OUT = 2000


def _silu(x):
    return x * (1.0 / (1.0 + jnp.exp(-x)))


def _dg0(a, b):
    return lax.dot_general(a, b, (((0,), (0,)), ((), ())),
                           preferred_element_type=jnp.float32)


def _node_body(nf_ref, na_ref, wsc_ref, w1_ref, sc_ref, nfo_ref):
    nf = nf_ref[...]
    na = na_ref[...]
    acc = na[:, 0:1] * jnp.dot(nf, wsc_ref[0], preferred_element_type=jnp.float32)
    for a in range(1, DA):
        acc = acc + na[:, a:a + 1] * jnp.dot(
            nf, wsc_ref[a], preferred_element_type=jnp.float32)
    sc_ref[...] = acc
    nfo_ref[...] = jnp.dot(nf, w1_ref[...], preferred_element_type=jnp.float32)


def _node_call(node_feats, node_attrs, wsc_r, w1):
    grid = (N // NB_NODE,)
    return pl.pallas_call(
        _node_body,
        grid=grid,
        in_specs=[
            pl.BlockSpec((NB_NODE, D), lambda i: (i, 0)),
            pl.BlockSpec((NB_NODE, DA), lambda i: (i, 0)),
            pl.BlockSpec((DA, D, D), lambda i: (0, 0, 0)),
            pl.BlockSpec((D, D), lambda i: (0, 0)),
        ],
        out_specs=[
            pl.BlockSpec((NB_NODE, D), lambda i: (i, 0)),
            pl.BlockSpec((NB_NODE, D), lambda i: (i, 0)),
        ],
        out_shape=[
            jax.ShapeDtypeStruct((N, D), jnp.float32),
            jax.ShapeDtypeStruct((N, D), jnp.float32),
        ],
    )(node_feats, node_attrs, wsc_r, w1)


def _edge_body(eft_ref, eat_ref, w0_ref, w1_ref, w2_ref, w3f_ref, g_ref):
    h = _silu(_dg0(w0_ref[...], eft_ref[...]))
    h = _silu(_dg0(w1_ref[...], h))
    h = _silu(_dg0(w2_ref[...], h))
    eat = eat_ref[...]
    hk = jnp.concatenate([eat[s:s + 1, :] * h for s in range(DS)], axis=0)
    g_ref[...] = _dg0(hk, w3f_ref[...])


def _edge_call(eft, eat, w0, w1, w2, w3f):
    grid = (E // EB_EDGE,)
    return pl.pallas_call(
        _edge_body,
        grid=grid,
        in_specs=[
            pl.BlockSpec((DE, EB_EDGE), lambda i: (0, i)),
            pl.BlockSpec((DS, EB_EDGE), lambda i: (0, i)),
            pl.BlockSpec((DE, H), lambda i: (0, 0)),
            pl.BlockSpec((H, H), lambda i: (0, 0)),
            pl.BlockSpec((H, H), lambda i: (0, 0)),
            pl.BlockSpec((DS * H, D), lambda i: (0, 0)),
        ],
        out_specs=pl.BlockSpec((EB_EDGE, D), lambda i: (i, 0)),
        out_shape=jax.ShapeDtypeStruct((E, D), jnp.float32),
    )(eft, eat, w0, w1, w2, w3f)


def _sc_scatter_body(nf_hbm, g_hbm, snd_hbm, rcv_hbm, zero_hbm, out_hbm,
                     sidx_all, ridx_all, rows0, grows0, rows1, grows1,
                     acc, sg0, sl0, sg1, sl1):
    c = lax.axis_index("c")
    s = lax.axis_index("s")
    wid = s * NC + c
    base = wid * EPW

    pltpu.sync_copy(zero_hbm, acc.at[pl.ds(s * NPS, NPS)])
    pltpu.sync_copy(snd_hbm.at[pl.ds(wid * EPW, EPW)], sidx_all)
    pltpu.sync_copy(rcv_hbm.at[wid], ridx_all)
    plsc.subcore_barrier()

    def issue(j, rows, grows, sg, sl):
        pltpu.async_copy(nf_hbm.at[sidx_all.at[pl.ds(j * CH, CH)]], rows, sg)
        pltpu.async_copy(g_hbm.at[pl.ds(base + j * CH, CH)], grows, sl)

    def wait(j, rows, grows, sg, sl):
        pltpu.make_async_copy(nf_hbm.at[sidx_all.at[pl.ds(j * CH, CH)]], rows, sg).wait()
        pltpu.make_async_copy(g_hbm.at[pl.ds(base + j * CH, CH)], grows, sl).wait()

    def mul(rows, grows):
        def mul8(i, _):
            for di in range(8):
                r = i * 8 + di
                for jj in range(D // 16):
                    sl_ = pl.ds(jj * 16, 16)
                    rows[r, sl_] = rows[r, sl_] * grows[r, sl_]
            return 0
        lax.fori_loop(0, CH // 8, mul8, 0)

    issue(0, rows0, grows0, sg0, sl0)
    issue(1, rows1, grows1, sg1, sl1)

    def pair(t, _):
        a = 2 * t
        wait(a, rows0, grows0, sg0, sl0)
        mul(rows0, grows0)
        pltpu.sync_copy(rows0, acc.at[ridx_all.at[a]], add=True)
        issue(a + 2, rows0, grows0, sg0, sl0)

        b = a + 1
        wait(b, rows1, grows1, sg1, sl1)
        mul(rows1, grows1)
        pltpu.sync_copy(rows1, acc.at[ridx_all.at[b]], add=True)

        @pl.when(t < (NCH - 1) // 2 - 1)
        def _():
            issue(b + 2, rows1, grows1, sg1, sl1)

        return 0

    lax.fori_loop(0, (NCH - 1) // 2, pair, 0)

    j = NCH - 1
    wait(j, rows0, grows0, sg0, sl0)
    mul(rows0, grows0)
    pltpu.sync_copy(rows0, acc.at[ridx_all.at[j]], add=True)

    plsc.subcore_barrier()
    pltpu.sync_copy(acc.at[pl.ds(s * NPS, NPS)],
                    out_hbm.at[pl.ds(c * NPAD + s * NPS, NPS)])


@functools.lru_cache(maxsize=1)
def _get_sc_scatter():
    mesh = plsc.VectorSubcoreMesh(core_axis_name="c", subcore_axis_name="s")
    return pl.kernel(
        _sc_scatter_body,
        mesh=mesh,
        out_type=jax.ShapeDtypeStruct((NC * NPAD, D), jnp.float32),
        scratch_types=[
            pltpu.VMEM((EPW,), jnp.int32),
            pltpu.VMEM((NCH, CH), jnp.int32),
            pltpu.VMEM((CH, D), jnp.float32),
            pltpu.VMEM((CH, D), jnp.float32),
            pltpu.VMEM((CH, D), jnp.float32),
            pltpu.VMEM((CH, D), jnp.float32),
            pltpu.VMEM_SHARED((NPAD, D), jnp.float32),
            pltpu.SemaphoreType.DMA,
            pltpu.SemaphoreType.DMA,
            pltpu.SemaphoreType.DMA,
            pltpu.SemaphoreType.DMA,
        ],
    )


def _out_body(p0_ref, p1_ref, w_ref, o_ref):
    o_ref[...] = jnp.dot(p0_ref[0] + p1_ref[0], w_ref[...],
                         preferred_element_type=jnp.float32)


def _out_call(partials, w_out):
    grid = (N // NB_OUT,)
    return pl.pallas_call(
        _out_body,
        grid=grid,
        in_specs=[
            pl.BlockSpec((1, NB_OUT, D), lambda i: (0, i, 0)),
            pl.BlockSpec((1, NB_OUT, D), lambda i: (1, i, 0)),
            pl.BlockSpec((D, D), lambda i: (0, 0)),
        ],
        out_specs=pl.BlockSpec((NB_OUT, D), lambda i: (i, 0)),
        out_shape=jax.ShapeDtypeStruct((N, D), jnp.float32),
    )(partials, partials, w_out)


def kernel(node_attrs, node_feats, edge_attrs, edge_feats, senders, receivers,
           W_sc, W1, W_m0, W_m1, W_m2, W_m3, W_out):
    wsc_r = W_sc.reshape(D, DA, D).transpose(1, 0, 2)
    w3f = (W_m3.reshape(H, D, DS).transpose(2, 0, 1).reshape(DS * H, D)
           * INV_SQRT_AVG)
    rcv2 = receivers.reshape(NW, NCH, CH)
    zeros = jnp.zeros((NPS, D), jnp.float32)

    sc, nf = _node_call(node_feats, node_attrs, wsc_r, W1)
    g = _edge_call(edge_feats.T, edge_attrs.T, W_m0, W_m1, W_m2, w3f)
    partials = _get_sc_scatter()(nf, g, senders, rcv2, zeros)
    message = _out_call(partials.reshape(NC, NPAD, D), W_out)
    return (message, sc)


# speedup vs baseline: 1.2698x; 1.0048x over previous
import functools

import jax
import jax.numpy as jnp
from jax import lax
from jax.experimental import pallas as pl
from jax.experimental.pallas import tpu as pltpu
from jax.experimental.pallas import tpu_sc as plsc

N = 10000
E = 160000
D = 128
DA = 16
DS = 4
DE = 16
H = 64
INV_SQRT_AVG = 0.25

NC = 2
NS = 16
NW = NC * NS
EPW = E // NW
CH = 40
E_SPLIT = 64000
EPW_A = E_SPLIT // NW
EPW_B = (E - E_SPLIT) // NW
NCH_A = EPW_A // CH
NCH_B = EPW_B // CH
NPAD = 10112
NPS = NPAD // NS

NB_NODE = 1000
EB_EDGE = 6400
NB_OUT = 2000


def _silu(x):
    return x * (1.0 / (1.0 + jnp.exp(-x)))


def _dg0(a, b):
    return lax.dot_general(a, b, (((0,), (0,)), ((), ())),
                           preferred_element_type=jnp.float32)


def _node_body(nf_ref, na_ref, wsc_ref, w1_ref, sc_ref, nfo_ref):
    nf = nf_ref[...]
    na = na_ref[...]
    acc = na[:, 0:1] * jnp.dot(nf, wsc_ref[0], preferred_element_type=jnp.float32)
    for a in range(1, DA):
        acc = acc + na[:, a:a + 1] * jnp.dot(
            nf, wsc_ref[a], preferred_element_type=jnp.float32)
    sc_ref[...] = acc
    nfo_ref[...] = jnp.dot(nf, w1_ref[...], preferred_element_type=jnp.float32)


def _node_call(node_feats, node_attrs, wsc_r, w1):
    grid = (N // NB_NODE,)
    return pl.pallas_call(
        _node_body,
        grid=grid,
        in_specs=[
            pl.BlockSpec((NB_NODE, D), lambda i: (i, 0)),
            pl.BlockSpec((NB_NODE, DA), lambda i: (i, 0)),
            pl.BlockSpec((DA, D, D), lambda i: (0, 0, 0)),
            pl.BlockSpec((D, D), lambda i: (0, 0)),
        ],
        out_specs=[
            pl.BlockSpec((NB_NODE, D), lambda i: (i, 0)),
            pl.BlockSpec((NB_NODE, D), lambda i: (i, 0)),
        ],
        out_shape=[
            jax.ShapeDtypeStruct((N, D), jnp.float32),
            jax.ShapeDtypeStruct((N, D), jnp.float32),
        ],
    )(node_feats, node_attrs, wsc_r, w1)


def _edge_body(eft_ref, eat_ref, w0_ref, w1_ref, w2_ref, w3f_ref, g_ref):
    h = _silu(_dg0(w0_ref[...], eft_ref[...]))
    h = _silu(_dg0(w1_ref[...], h))
    h = _silu(_dg0(w2_ref[...], h))
    eat = eat_ref[...]
    hk = jnp.concatenate([eat[s:s + 1, :] * h for s in range(DS)], axis=0)
    g_ref[...] = _dg0(hk, w3f_ref[...])


def _edge_call(eft, eat, w0, w1, w2, w3f, e_lo, e_hi):
    nb = (e_hi - e_lo) // EB_EDGE
    off = e_lo // EB_EDGE
    return pl.pallas_call(
        _edge_body,
        grid=(nb,),
        in_specs=[
            pl.BlockSpec((DE, EB_EDGE), lambda i: (0, i + off)),
            pl.BlockSpec((DS, EB_EDGE), lambda i: (0, i + off)),
            pl.BlockSpec((DE, H), lambda i: (0, 0)),
            pl.BlockSpec((H, H), lambda i: (0, 0)),
            pl.BlockSpec((H, H), lambda i: (0, 0)),
            pl.BlockSpec((DS * H, D), lambda i: (0, 0)),
        ],
        out_specs=pl.BlockSpec((EB_EDGE, D), lambda i: (i, 0)),
        out_shape=jax.ShapeDtypeStruct((e_hi - e_lo, D), jnp.float32),
    )(eft, eat, w0, w1, w2, w3f)


def _make_sc_body(edge_base, epw_p, nch_p):
    pairs = nch_p // 2
    odd = nch_p % 2

    def body(nf_hbm, g_hbm, snd_hbm, rcv_hbm, zero_hbm, out_hbm,
             sidx_all, ridx_all, rows0, grows0, rows1, grows1,
             acc, sg0, sl0, sg1, sl1):
        c = lax.axis_index("c")
        s = lax.axis_index("s")
        wid = s * NC + c
        gbase = wid * epw_p

        pltpu.sync_copy(zero_hbm, acc.at[pl.ds(s * NPS, NPS)])
        pltpu.sync_copy(snd_hbm.at[pl.ds(edge_base + wid * epw_p, epw_p)],
                        sidx_all)
        pltpu.sync_copy(rcv_hbm.at[wid], ridx_all)
        plsc.subcore_barrier()

        def issue(j, rows, grows, sg, sl):
            pltpu.async_copy(nf_hbm.at[sidx_all.at[pl.ds(j * CH, CH)]], rows, sg)
            pltpu.async_copy(g_hbm.at[pl.ds(gbase + j * CH, CH)], grows, sl)

        def wait(j, rows, grows, sg, sl):
            pltpu.make_async_copy(
                nf_hbm.at[sidx_all.at[pl.ds(j * CH, CH)]], rows, sg).wait()
            pltpu.make_async_copy(
                g_hbm.at[pl.ds(gbase + j * CH, CH)], grows, sl).wait()

        def mul(rows, grows):
            def mul8(i, _):
                for di in range(8):
                    r = i * 8 + di
                    for jj in range(D // 16):
                        sl_ = pl.ds(jj * 16, 16)
                        rows[r, sl_] = rows[r, sl_] * grows[r, sl_]
                return 0
            lax.fori_loop(0, CH // 8, mul8, 0)

        issue(0, rows0, grows0, sg0, sl0)
        issue(1, rows1, grows1, sg1, sl1)

        def pair(t, _):
            a = 2 * t
            wait(a, rows0, grows0, sg0, sl0)
            mul(rows0, grows0)
            pltpu.sync_copy(rows0, acc.at[ridx_all.at[a]], add=True)
            if odd:
                issue(a + 2, rows0, grows0, sg0, sl0)
            else:
                @pl.when(t < pairs - 1)
                def _():
                    issue(a + 2, rows0, grows0, sg0, sl0)

            b = a + 1
            wait(b, rows1, grows1, sg1, sl1)
            mul(rows1, grows1)
            pltpu.sync_copy(rows1, acc.at[ridx_all.at[b]], add=True)

            @pl.when(t < pairs - 1)
            def _():
                issue(b + 2, rows1, grows1, sg1, sl1)

            return 0

        lax.fori_loop(0, pairs, pair, 0)

        if odd:
            j = nch_p - 1
            wait(j, rows0, grows0, sg0, sl0)
            mul(rows0, grows0)
            pltpu.sync_copy(rows0, acc.at[ridx_all.at[j]], add=True)

        plsc.subcore_barrier()
        pltpu.sync_copy(acc.at[pl.ds(s * NPS, NPS)],
                        out_hbm.at[pl.ds(c * NPAD + s * NPS, NPS)])

    return body


@functools.lru_cache(maxsize=4)
def _get_sc_scatter(edge_base, epw_p, nch_p):
    mesh = plsc.VectorSubcoreMesh(core_axis_name="c", subcore_axis_name="s")
    return pl.kernel(
        _make_sc_body(edge_base, epw_p, nch_p),
        mesh=mesh,
        out_type=jax.ShapeDtypeStruct((NC * NPAD, D), jnp.float32),
        scratch_types=[
            pltpu.VMEM((epw_p,), jnp.int32),
            pltpu.VMEM((nch_p, CH), jnp.int32),
            pltpu.VMEM((CH, D), jnp.float32),
            pltpu.VMEM((CH, D), jnp.float32),
            pltpu.VMEM((CH, D), jnp.float32),
            pltpu.VMEM((CH, D), jnp.float32),
            pltpu.VMEM_SHARED((NPAD, D), jnp.float32),
            pltpu.SemaphoreType.DMA,
            pltpu.SemaphoreType.DMA,
            pltpu.SemaphoreType.DMA,
            pltpu.SemaphoreType.DMA,
        ],
    )


def _out_body(a0_ref, a1_ref, b0_ref, b1_ref, w_ref, o_ref):
    p = (a0_ref[0] + a1_ref[0]) + (b0_ref[0] + b1_ref[0])
    o_ref[...] = jnp.dot(p, w_ref[...], preferred_element_type=jnp.float32)


def _out_call(pa, pb, w_out):
    grid = (N // NB_OUT,)
    specs = [
        pl.BlockSpec((1, NB_OUT, D), lambda i: (0, i, 0)),
        pl.BlockSpec((1, NB_OUT, D), lambda i: (1, i, 0)),
    ]
    return pl.pallas_call(
        _out_body,
        grid=grid,
        in_specs=specs + specs + [pl.BlockSpec((D, D), lambda i: (0, 0))],
        out_specs=pl.BlockSpec((NB_OUT, D), lambda i: (i, 0)),
        out_shape=jax.ShapeDtypeStruct((N, D), jnp.float32),
    )(pa, pa, pb, pb, w_out)


def kernel(node_attrs, node_feats, edge_attrs, edge_feats, senders, receivers,
           W_sc, W1, W_m0, W_m1, W_m2, W_m3, W_out):
    wsc_r = W_sc.reshape(D, DA, D).transpose(1, 0, 2)
    w3f = (W_m3.reshape(H, D, DS).transpose(2, 0, 1).reshape(DS * H, D)
           * INV_SQRT_AVG)
    zeros = jnp.zeros((NPS, D), jnp.float32)
    rcv_a = receivers[:E_SPLIT].reshape(NW, NCH_A, CH)
    rcv_b = receivers[E_SPLIT:].reshape(NW, NCH_B, CH)

    sc, nf = _node_call(node_feats, node_attrs, wsc_r, W1)
    eft, eat = edge_feats.T, edge_attrs.T
    g_a = _edge_call(eft, eat, W_m0, W_m1, W_m2, w3f, 0, E_SPLIT)
    p_a = _get_sc_scatter(0, EPW_A, NCH_A)(nf, g_a, senders, rcv_a, zeros)
    g_b = _edge_call(eft, eat, W_m0, W_m1, W_m2, w3f, E_SPLIT, E)
    p_b = _get_sc_scatter(E_SPLIT, EPW_B, NCH_B)(nf, g_b, senders, rcv_b, zeros)
    message = _out_call(p_a.reshape(NC, NPAD, D), p_b.reshape(NC, NPAD, D), W_out)
    return (message, sc)


# speedup vs baseline: 1.3071x; 1.0294x over previous
import functools

import jax
import jax.numpy as jnp
from jax import lax
from jax.experimental import pallas as pl
from jax.experimental.pallas import tpu as pltpu
from jax.experimental.pallas import tpu_sc as plsc

N = 10000
E = 160000
D = 128
DA = 16
DS = 4
DE = 16
H = 64
INV_SQRT_AVG = 0.25

NC = 2
NS = 16
NW = NC * NS
EPW = E // NW
CH = 40
E_SPLIT = 64000
EPW_A = E_SPLIT // NW
EPW_B = (E - E_SPLIT) // NW
NCH_A = EPW_A // CH
NCH_B = EPW_B // CH
NPAD = 10112
NPS = NPAD // NS

NB_NODE = 400
EB_EDGE = 6400
NB_OUT = 2000


def _silu(x):
    return x * (1.0 / (1.0 + jnp.exp(-x)))


def _dg0(a, b):
    return lax.dot_general(a, b, (((0,), (0,)), ((), ())),
                           preferred_element_type=jnp.float32)


def _node_body(nf_ref, na_ref, wsc_ref, w1_ref, sc_ref, nfo_ref):
    nf = nf_ref[...]
    na = na_ref[...]
    res = jnp.dot(nf, wsc_ref[...], preferred_element_type=jnp.float32)
    acc = na[:, 0:1] * res[:, 0:D]
    for a in range(1, DA):
        acc = acc + na[:, a:a + 1] * res[:, a * D:(a + 1) * D]
    sc_ref[...] = acc
    nfo_ref[...] = jnp.dot(nf, w1_ref[...], preferred_element_type=jnp.float32)


def _node_call(node_feats, node_attrs, wsc_r, w1):
    grid = (N // NB_NODE,)
    return pl.pallas_call(
        _node_body,
        grid=grid,
        in_specs=[
            pl.BlockSpec((NB_NODE, D), lambda i: (i, 0)),
            pl.BlockSpec((NB_NODE, DA), lambda i: (i, 0)),
            pl.BlockSpec((D, DA * D), lambda i: (0, 0)),
            pl.BlockSpec((D, D), lambda i: (0, 0)),
        ],
        out_specs=[
            pl.BlockSpec((NB_NODE, D), lambda i: (i, 0)),
            pl.BlockSpec((NB_NODE, D), lambda i: (i, 0)),
        ],
        out_shape=[
            jax.ShapeDtypeStruct((N, D), jnp.float32),
            jax.ShapeDtypeStruct((N, D), jnp.float32),
        ],
    )(node_feats, node_attrs, wsc_r, w1)


def _edge_body(eft_ref, eat_ref, w0_ref, w1_ref, w2_ref, w3f_ref, g_ref):
    h = _silu(_dg0(w0_ref[...], eft_ref[...]))
    h = _silu(_dg0(w1_ref[...], h))
    h = _silu(_dg0(w2_ref[...], h))
    eat = eat_ref[...]
    hk = jnp.concatenate([eat[s:s + 1, :] * h for s in range(DS)], axis=0)
    g_ref[...] = _dg0(hk, w3f_ref[...])


def _edge_call(eft, eat, w0, w1, w2, w3f, e_lo, e_hi):
    nb = (e_hi - e_lo) // EB_EDGE
    off = e_lo // EB_EDGE
    return pl.pallas_call(
        _edge_body,
        grid=(nb,),
        in_specs=[
            pl.BlockSpec((DE, EB_EDGE), lambda i: (0, i + off)),
            pl.BlockSpec((DS, EB_EDGE), lambda i: (0, i + off)),
            pl.BlockSpec((DE, H), lambda i: (0, 0)),
            pl.BlockSpec((H, H), lambda i: (0, 0)),
            pl.BlockSpec((H, H), lambda i: (0, 0)),
            pl.BlockSpec((DS * H, D), lambda i: (0, 0)),
        ],
        out_specs=pl.BlockSpec((EB_EDGE, D), lambda i: (i, 0)),
        out_shape=jax.ShapeDtypeStruct((e_hi - e_lo, D), jnp.float32),
    )(eft, eat, w0, w1, w2, w3f)


def _make_sc_body(edge_base, epw_p, nch_p):
    pairs = nch_p // 2
    odd = nch_p % 2

    def body(nf_hbm, g_hbm, snd_hbm, rcv_hbm, zero_hbm, out_hbm,
             sidx_all, ridx_all, rows0, grows0, rows1, grows1,
             acc, sg0, sl0, sg1, sl1):
        c = lax.axis_index("c")
        s = lax.axis_index("s")
        wid = s * NC + c
        gbase = wid * epw_p

        pltpu.sync_copy(zero_hbm, acc.at[pl.ds(s * NPS, NPS)])
        pltpu.sync_copy(snd_hbm.at[pl.ds(edge_base + wid * epw_p, epw_p)],
                        sidx_all)
        pltpu.sync_copy(rcv_hbm.at[wid], ridx_all)
        plsc.subcore_barrier()

        def issue(j, rows, grows, sg, sl):
            pltpu.async_copy(nf_hbm.at[sidx_all.at[pl.ds(j * CH, CH)]], rows, sg)
            pltpu.async_copy(g_hbm.at[pl.ds(gbase + j * CH, CH)], grows, sl)

        def wait(j, rows, grows, sg, sl):
            pltpu.make_async_copy(
                nf_hbm.at[sidx_all.at[pl.ds(j * CH, CH)]], rows, sg).wait()
            pltpu.make_async_copy(
                g_hbm.at[pl.ds(gbase + j * CH, CH)], grows, sl).wait()

        def mul(rows, grows):
            def mul8(i, _):
                for di in range(8):
                    r = i * 8 + di
                    for jj in range(D // 16):
                        sl_ = pl.ds(jj * 16, 16)
                        rows[r, sl_] = rows[r, sl_] * grows[r, sl_]
                return 0
            lax.fori_loop(0, CH // 8, mul8, 0)

        issue(0, rows0, grows0, sg0, sl0)
        issue(1, rows1, grows1, sg1, sl1)

        def pair(t, _):
            a = 2 * t
            wait(a, rows0, grows0, sg0, sl0)
            mul(rows0, grows0)
            pltpu.sync_copy(rows0, acc.at[ridx_all.at[a]], add=True)
            if odd:
                issue(a + 2, rows0, grows0, sg0, sl0)
            else:
                @pl.when(t < pairs - 1)
                def _():
                    issue(a + 2, rows0, grows0, sg0, sl0)

            b = a + 1
            wait(b, rows1, grows1, sg1, sl1)
            mul(rows1, grows1)
            pltpu.sync_copy(rows1, acc.at[ridx_all.at[b]], add=True)

            @pl.when(t < pairs - 1)
            def _():
                issue(b + 2, rows1, grows1, sg1, sl1)

            return 0

        lax.fori_loop(0, pairs, pair, 0)

        if odd:
            j = nch_p - 1
            wait(j, rows0, grows0, sg0, sl0)
            mul(rows0, grows0)
            pltpu.sync_copy(rows0, acc.at[ridx_all.at[j]], add=True)

        plsc.subcore_barrier()
        pltpu.sync_copy(acc.at[pl.ds(s * NPS, NPS)],
                        out_hbm.at[pl.ds(c * NPAD + s * NPS, NPS)])

    return body


@functools.lru_cache(maxsize=4)
def _get_sc_scatter(edge_base, epw_p, nch_p):
    mesh = plsc.VectorSubcoreMesh(core_axis_name="c", subcore_axis_name="s")
    return pl.kernel(
        _make_sc_body(edge_base, epw_p, nch_p),
        mesh=mesh,
        out_type=jax.ShapeDtypeStruct((NC * NPAD, D), jnp.float32),
        scratch_types=[
            pltpu.VMEM((epw_p,), jnp.int32),
            pltpu.VMEM((nch_p, CH), jnp.int32),
            pltpu.VMEM((CH, D), jnp.float32),
            pltpu.VMEM((CH, D), jnp.float32),
            pltpu.VMEM((CH, D), jnp.float32),
            pltpu.VMEM((CH, D), jnp.float32),
            pltpu.VMEM_SHARED((NPAD, D), jnp.float32),
            pltpu.SemaphoreType.DMA,
            pltpu.SemaphoreType.DMA,
            pltpu.SemaphoreType.DMA,
            pltpu.SemaphoreType.DMA,
        ],
    )


def _out_body(a0_ref, a1_ref, b0_ref, b1_ref, w_ref, o_ref):
    p = (a0_ref[0] + a1_ref[0]) + (b0_ref[0] + b1_ref[0])
    o_ref[...] = jnp.dot(p, w_ref[...], preferred_element_type=jnp.float32)


def _out_call(pa, pb, w_out):
    grid = (N // NB_OUT,)
    specs = [
        pl.BlockSpec((1, NB_OUT, D), lambda i: (0, i, 0)),
        pl.BlockSpec((1, NB_OUT, D), lambda i: (1, i, 0)),
    ]
    return pl.pallas_call(
        _out_body,
        grid=grid,
        in_specs=specs + specs + [pl.BlockSpec((D, D), lambda i: (0, 0))],
        out_specs=pl.BlockSpec((NB_OUT, D), lambda i: (i, 0)),
        out_shape=jax.ShapeDtypeStruct((N, D), jnp.float32),
    )(pa, pa, pb, pb, w_out)


def kernel(node_attrs, node_feats, edge_attrs, edge_feats, senders, receivers,
           W_sc, W1, W_m0, W_m1, W_m2, W_m3, W_out):
    wsc_r = W_sc.reshape(D, DA * D)
    w3f = (W_m3.reshape(H, D, DS).transpose(2, 0, 1).reshape(DS * H, D)
           * INV_SQRT_AVG)
    zeros = jnp.zeros((NPS, D), jnp.float32)
    rcv_a = receivers[:E_SPLIT].reshape(NW, NCH_A, CH)
    rcv_b = receivers[E_SPLIT:].reshape(NW, NCH_B, CH)

    sc, nf = _node_call(node_feats, node_attrs, wsc_r, W1)
    eft, eat = edge_feats.T, edge_attrs.T
    g_a = _edge_call(eft, eat, W_m0, W_m1, W_m2, w3f, 0, E_SPLIT)
    p_a = _get_sc_scatter(0, EPW_A, NCH_A)(nf, g_a, senders, rcv_a, zeros)
    w0_b = W_m0 + g_a[0, 0] * 0.0
    g_b = _edge_call(eft, eat, w0_b, W_m1, W_m2, w3f, E_SPLIT, E)
    p_b = _get_sc_scatter(E_SPLIT, EPW_B, NCH_B)(nf, g_b, senders, rcv_b, zeros)
    message = _out_call(p_a.reshape(NC, NPAD, D), p_b.reshape(NC, NPAD, D), W_out)
    return (message, sc)


# speedup vs baseline: 1.4590x; 1.1162x over previous
import functools

import jax
import jax.numpy as jnp
from jax import lax
from jax.experimental import pallas as pl
from jax.experimental.pallas import tpu as pltpu
from jax.experimental.pallas import tpu_sc as plsc

N = 10000
E = 160000
D = 128
DA = 16
DS = 4
DE = 16
H = 64
INV_SQRT_AVG = 0.25

NC = 2
NS = 16
NW = NC * NS
EPW = E // NW
CH = 40
E_SPLIT = 64000
EPW_A = E_SPLIT // NW
EPW_B = (E - E_SPLIT) // NW
NCH_A = EPW_A // CH
NCH_B = EPW_B // CH
NPAD = 10112
NPS = NPAD // NS

NB_NODE = 400
EB_EDGE = 6400
NB_OUT = 2000


def _silu(x):
    return x * (1.0 / (1.0 + jnp.exp(-x)))


def _dg0(a, b):
    return lax.dot_general(a, b, (((0,), (0,)), ((), ())),
                           preferred_element_type=jnp.float32)


def _sc_node_body(nf_ref, na_ref, wsc_ref, sc_ref):
    nf = nf_ref[...]
    na = na_ref[...]
    res = jnp.dot(nf, wsc_ref[...], preferred_element_type=jnp.float32)
    acc = na[:, 0:1] * res[:, 0:D]
    for a in range(1, DA):
        acc = acc + na[:, a:a + 1] * res[:, a * D:(a + 1) * D]
    sc_ref[...] = acc


def _sc_node_call(node_feats, node_attrs, wsc_r):
    grid = (N // NB_NODE,)
    return pl.pallas_call(
        _sc_node_body,
        grid=grid,
        in_specs=[
            pl.BlockSpec((NB_NODE, D), lambda i: (i, 0)),
            pl.BlockSpec((NB_NODE, DA), lambda i: (i, 0)),
            pl.BlockSpec((D, DA * D), lambda i: (0, 0)),
        ],
        out_specs=pl.BlockSpec((NB_NODE, D), lambda i: (i, 0)),
        out_shape=jax.ShapeDtypeStruct((N, D), jnp.float32),
    )(node_feats, node_attrs, wsc_r)


def _nf_body(nf_ref, w1_ref, nfo_ref):
    nfo_ref[...] = jnp.dot(nf_ref[...], w1_ref[...],
                           preferred_element_type=jnp.float32)


def _nf_call(node_feats, w1):
    grid = (N // NB_OUT,)
    return pl.pallas_call(
        _nf_body,
        grid=grid,
        in_specs=[
            pl.BlockSpec((NB_OUT, D), lambda i: (i, 0)),
            pl.BlockSpec((D, D), lambda i: (0, 0)),
        ],
        out_specs=pl.BlockSpec((NB_OUT, D), lambda i: (i, 0)),
        out_shape=jax.ShapeDtypeStruct((N, D), jnp.float32),
    )(node_feats, w1)


def _edge_body(eft_ref, eat_ref, w0_ref, w1_ref, w2_ref, w3f_ref, g_ref):
    h = _silu(_dg0(w0_ref[...], eft_ref[...]))
    h = _silu(_dg0(w1_ref[...], h))
    h = _silu(_dg0(w2_ref[...], h))
    eat = eat_ref[...]
    hk = jnp.concatenate([eat[s:s + 1, :] * h for s in range(DS)], axis=0)
    g_ref[...] = _dg0(hk, w3f_ref[...])


def _edge_call(eft, eat, w0, w1, w2, w3f, e_lo, e_hi):
    nb = (e_hi - e_lo) // EB_EDGE
    off = e_lo // EB_EDGE
    return pl.pallas_call(
        _edge_body,
        grid=(nb,),
        in_specs=[
            pl.BlockSpec((DE, EB_EDGE), lambda i: (0, i + off)),
            pl.BlockSpec((DS, EB_EDGE), lambda i: (0, i + off)),
            pl.BlockSpec((DE, H), lambda i: (0, 0)),
            pl.BlockSpec((H, H), lambda i: (0, 0)),
            pl.BlockSpec((H, H), lambda i: (0, 0)),
            pl.BlockSpec((DS * H, D), lambda i: (0, 0)),
        ],
        out_specs=pl.BlockSpec((EB_EDGE, D), lambda i: (i, 0)),
        out_shape=jax.ShapeDtypeStruct((e_hi - e_lo, D), jnp.float32),
    )(eft, eat, w0, w1, w2, w3f)


def _make_sc_body(edge_base, epw_p, nch_p):
    pairs = nch_p // 2
    odd = nch_p % 2

    def body(nf_hbm, g_hbm, snd_hbm, rcv_hbm, zero_hbm, out_hbm,
             sidx_all, ridx_all, rows0, grows0, rows1, grows1,
             acc, sg0, sl0, sg1, sl1):
        c = lax.axis_index("c")
        s = lax.axis_index("s")
        wid = s * NC + c
        gbase = wid * epw_p

        pltpu.sync_copy(zero_hbm, acc.at[pl.ds(s * NPS, NPS)])
        pltpu.sync_copy(snd_hbm.at[pl.ds(edge_base + wid * epw_p, epw_p)],
                        sidx_all)
        pltpu.sync_copy(rcv_hbm.at[wid], ridx_all)
        plsc.subcore_barrier()

        def issue(j, rows, grows, sg, sl):
            pltpu.async_copy(nf_hbm.at[sidx_all.at[pl.ds(j * CH, CH)]], rows, sg)
            pltpu.async_copy(g_hbm.at[pl.ds(gbase + j * CH, CH)], grows, sl)

        def wait(j, rows, grows, sg, sl):
            pltpu.make_async_copy(
                nf_hbm.at[sidx_all.at[pl.ds(j * CH, CH)]], rows, sg).wait()
            pltpu.make_async_copy(
                g_hbm.at[pl.ds(gbase + j * CH, CH)], grows, sl).wait()

        def mul(rows, grows):
            def mul8(i, _):
                for di in range(8):
                    r = i * 8 + di
                    for jj in range(D // 16):
                        sl_ = pl.ds(jj * 16, 16)
                        rows[r, sl_] = rows[r, sl_] * grows[r, sl_]
                return 0
            lax.fori_loop(0, CH // 8, mul8, 0)

        issue(0, rows0, grows0, sg0, sl0)
        issue(1, rows1, grows1, sg1, sl1)

        def pair(t, _):
            a = 2 * t
            wait(a, rows0, grows0, sg0, sl0)
            mul(rows0, grows0)
            pltpu.sync_copy(rows0, acc.at[ridx_all.at[a]], add=True)
            if odd:
                issue(a + 2, rows0, grows0, sg0, sl0)
            else:
                @pl.when(t < pairs - 1)
                def _():
                    issue(a + 2, rows0, grows0, sg0, sl0)

            b = a + 1
            wait(b, rows1, grows1, sg1, sl1)
            mul(rows1, grows1)
            pltpu.sync_copy(rows1, acc.at[ridx_all.at[b]], add=True)

            @pl.when(t < pairs - 1)
            def _():
                issue(b + 2, rows1, grows1, sg1, sl1)

            return 0

        lax.fori_loop(0, pairs, pair, 0)

        if odd:
            j = nch_p - 1
            wait(j, rows0, grows0, sg0, sl0)
            mul(rows0, grows0)
            pltpu.sync_copy(rows0, acc.at[ridx_all.at[j]], add=True)

        plsc.subcore_barrier()
        pltpu.sync_copy(acc.at[pl.ds(s * NPS, NPS)],
                        out_hbm.at[pl.ds(c * NPAD + s * NPS, NPS)])

    return body


@functools.lru_cache(maxsize=4)
def _get_sc_scatter(edge_base, epw_p, nch_p):
    mesh = plsc.VectorSubcoreMesh(core_axis_name="c", subcore_axis_name="s")
    return pl.kernel(
        _make_sc_body(edge_base, epw_p, nch_p),
        mesh=mesh,
        out_type=jax.ShapeDtypeStruct((NC * NPAD, D), jnp.float32),
        scratch_types=[
            pltpu.VMEM((epw_p,), jnp.int32),
            pltpu.VMEM((nch_p, CH), jnp.int32),
            pltpu.VMEM((CH, D), jnp.float32),
            pltpu.VMEM((CH, D), jnp.float32),
            pltpu.VMEM((CH, D), jnp.float32),
            pltpu.VMEM((CH, D), jnp.float32),
            pltpu.VMEM_SHARED((NPAD, D), jnp.float32),
            pltpu.SemaphoreType.DMA,
            pltpu.SemaphoreType.DMA,
            pltpu.SemaphoreType.DMA,
            pltpu.SemaphoreType.DMA,
        ],
    )


def _out_body(a0_ref, a1_ref, b0_ref, b1_ref, w_ref, o_ref):
    p = (a0_ref[0] + a1_ref[0]) + (b0_ref[0] + b1_ref[0])
    o_ref[...] = jnp.dot(p, w_ref[...], preferred_element_type=jnp.float32)


def _out_call(pa, pb, w_out):
    grid = (N // NB_OUT,)
    specs = [
        pl.BlockSpec((1, NB_OUT, D), lambda i: (0, i, 0)),
        pl.BlockSpec((1, NB_OUT, D), lambda i: (1, i, 0)),
    ]
    return pl.pallas_call(
        _out_body,
        grid=grid,
        in_specs=specs + specs + [pl.BlockSpec((D, D), lambda i: (0, 0))],
        out_specs=pl.BlockSpec((NB_OUT, D), lambda i: (i, 0)),
        out_shape=jax.ShapeDtypeStruct((N, D), jnp.float32),
    )(pa, pa, pb, pb, w_out)


def kernel(node_attrs, node_feats, edge_attrs, edge_feats, senders, receivers,
           W_sc, W1, W_m0, W_m1, W_m2, W_m3, W_out):
    wsc_r = W_sc.reshape(D, DA * D)
    w3f = (W_m3.reshape(H, D, DS).transpose(2, 0, 1).reshape(DS * H, D)
           * INV_SQRT_AVG)
    zeros = jnp.zeros((NPS, D), jnp.float32)
    rcv_a = receivers[:E_SPLIT].reshape(NW, NCH_A, CH)
    rcv_b = receivers[E_SPLIT:].reshape(NW, NCH_B, CH)

    nf = _nf_call(node_feats, W1)
    eft, eat = edge_feats.T, edge_attrs.T
    g_a = _edge_call(eft, eat, W_m0, W_m1, W_m2, w3f, 0, E_SPLIT)
    p_a = _get_sc_scatter(0, EPW_A, NCH_A)(nf, g_a, senders, rcv_a, zeros)
    w0_b = W_m0 + g_a[0, 0] * 0.0
    g_b = _edge_call(eft, eat, w0_b, W_m1, W_m2, w3f, E_SPLIT, E)
    p_b = _get_sc_scatter(E_SPLIT, EPW_B, NCH_B)(nf, g_b, senders, rcv_b, zeros)
    wsc_dep = wsc_r + g_b[0, 0] * 0.0
    sc = _sc_node_call(node_feats, node_attrs, wsc_dep)
    message = _out_call(p_a.reshape(NC, NPAD, D), p_b.reshape(NC, NPAD, D), W_out)
    return (message, sc)


# speedup vs baseline: 1.4854x; 1.0182x over previous
import functools

import jax
import jax.numpy as jnp
from jax import lax
from jax.experimental import pallas as pl
from jax.experimental.pallas import tpu as pltpu
from jax.experimental.pallas import tpu_sc as plsc

N = 10000
E = 160000
D = 128
DA = 16
DS = 4
DE = 16
H = 64
INV_SQRT_AVG = 0.25

NC = 2
NS = 16
NW = NC * NS
EPW = E // NW
CH = 40
E_SPLIT = 44800
EPW_A = E_SPLIT // NW
EPW_B = (E - E_SPLIT) // NW
NCH_A = EPW_A // CH
NCH_B = EPW_B // CH
NPAD = 10112
NPS = NPAD // NS

NB_NODE = 400
EB_EDGE = 6400
NB_OUT = 2000


def _silu(x):
    return x * (1.0 / (1.0 + jnp.exp(-x)))


def _dg0(a, b):
    return lax.dot_general(a, b, (((0,), (0,)), ((), ())),
                           preferred_element_type=jnp.float32)


def _sc_node_body(nf_ref, na_ref, wsc_ref, sc_ref):
    nf = nf_ref[...]
    na = na_ref[...]
    res = jnp.dot(nf, wsc_ref[...], preferred_element_type=jnp.float32)
    acc = na[:, 0:1] * res[:, 0:D]
    for a in range(1, DA):
        acc = acc + na[:, a:a + 1] * res[:, a * D:(a + 1) * D]
    sc_ref[...] = acc


def _sc_node_call(node_feats, node_attrs, wsc_r):
    grid = (N // NB_NODE,)
    return pl.pallas_call(
        _sc_node_body,
        grid=grid,
        in_specs=[
            pl.BlockSpec((NB_NODE, D), lambda i: (i, 0)),
            pl.BlockSpec((NB_NODE, DA), lambda i: (i, 0)),
            pl.BlockSpec((D, DA * D), lambda i: (0, 0)),
        ],
        out_specs=pl.BlockSpec((NB_NODE, D), lambda i: (i, 0)),
        out_shape=jax.ShapeDtypeStruct((N, D), jnp.float32),
    )(node_feats, node_attrs, wsc_r)


def _nf_body(nf_ref, w1_ref, nfo_ref):
    nfo_ref[...] = jnp.dot(nf_ref[...], w1_ref[...],
                           preferred_element_type=jnp.float32)


def _nf_call(node_feats, w1):
    grid = (N // NB_OUT,)
    return pl.pallas_call(
        _nf_body,
        grid=grid,
        in_specs=[
            pl.BlockSpec((NB_OUT, D), lambda i: (i, 0)),
            pl.BlockSpec((D, D), lambda i: (0, 0)),
        ],
        out_specs=pl.BlockSpec((NB_OUT, D), lambda i: (i, 0)),
        out_shape=jax.ShapeDtypeStruct((N, D), jnp.float32),
    )(node_feats, w1)


def _edge_body(eft_ref, eat_ref, w0_ref, w1_ref, w2_ref, w3f_ref, g_ref):
    h = _silu(_dg0(w0_ref[...], eft_ref[...]))
    h = _silu(_dg0(w1_ref[...], h))
    h = _silu(_dg0(w2_ref[...], h))
    eat = eat_ref[...]
    hk = jnp.concatenate([eat[s:s + 1, :] * h for s in range(DS)], axis=0)
    g_ref[...] = _dg0(hk, w3f_ref[...])


def _edge_call(eft, eat, w0, w1, w2, w3f, e_lo, e_hi):
    nb = (e_hi - e_lo) // EB_EDGE
    off = e_lo // EB_EDGE
    return pl.pallas_call(
        _edge_body,
        grid=(nb,),
        in_specs=[
            pl.BlockSpec((DE, EB_EDGE), lambda i: (0, i + off)),
            pl.BlockSpec((DS, EB_EDGE), lambda i: (0, i + off)),
            pl.BlockSpec((DE, H), lambda i: (0, 0)),
            pl.BlockSpec((H, H), lambda i: (0, 0)),
            pl.BlockSpec((H, H), lambda i: (0, 0)),
            pl.BlockSpec((DS * H, D), lambda i: (0, 0)),
        ],
        out_specs=pl.BlockSpec((EB_EDGE, D), lambda i: (i, 0)),
        out_shape=jax.ShapeDtypeStruct((e_hi - e_lo, D), jnp.float32),
    )(eft, eat, w0, w1, w2, w3f)


def _make_sc_body(edge_base, epw_p, nch_p):
    pairs = nch_p // 2
    odd = nch_p % 2

    def body(nf_hbm, g_hbm, snd_hbm, rcv_hbm, zero_hbm, out_hbm,
             sidx_all, ridx_all, rows0, grows0, rows1, grows1,
             acc, sg0, sl0, sg1, sl1):
        c = lax.axis_index("c")
        s = lax.axis_index("s")
        wid = s * NC + c
        gbase = wid * epw_p

        pltpu.sync_copy(zero_hbm, acc.at[pl.ds(s * NPS, NPS)])
        pltpu.sync_copy(snd_hbm.at[pl.ds(edge_base + wid * epw_p, epw_p)],
                        sidx_all)
        pltpu.sync_copy(rcv_hbm.at[wid], ridx_all)
        plsc.subcore_barrier()

        def issue(j, rows, grows, sg, sl):
            pltpu.async_copy(nf_hbm.at[sidx_all.at[pl.ds(j * CH, CH)]], rows, sg)
            pltpu.async_copy(g_hbm.at[pl.ds(gbase + j * CH, CH)], grows, sl)

        def wait(j, rows, grows, sg, sl):
            pltpu.make_async_copy(
                nf_hbm.at[sidx_all.at[pl.ds(j * CH, CH)]], rows, sg).wait()
            pltpu.make_async_copy(
                g_hbm.at[pl.ds(gbase + j * CH, CH)], grows, sl).wait()

        def mul(rows, grows):
            def mul8(i, _):
                for di in range(8):
                    r = i * 8 + di
                    for jj in range(D // 16):
                        sl_ = pl.ds(jj * 16, 16)
                        rows[r, sl_] = rows[r, sl_] * grows[r, sl_]
                return 0
            lax.fori_loop(0, CH // 8, mul8, 0)

        issue(0, rows0, grows0, sg0, sl0)
        issue(1, rows1, grows1, sg1, sl1)

        def pair(t, _):
            a = 2 * t
            wait(a, rows0, grows0, sg0, sl0)
            mul(rows0, grows0)
            pltpu.sync_copy(rows0, acc.at[ridx_all.at[a]], add=True)
            if odd:
                issue(a + 2, rows0, grows0, sg0, sl0)
            else:
                @pl.when(t < pairs - 1)
                def _():
                    issue(a + 2, rows0, grows0, sg0, sl0)

            b = a + 1
            wait(b, rows1, grows1, sg1, sl1)
            mul(rows1, grows1)
            pltpu.sync_copy(rows1, acc.at[ridx_all.at[b]], add=True)

            @pl.when(t < pairs - 1)
            def _():
                issue(b + 2, rows1, grows1, sg1, sl1)

            return 0

        lax.fori_loop(0, pairs, pair, 0)

        if odd:
            j = nch_p - 1
            wait(j, rows0, grows0, sg0, sl0)
            mul(rows0, grows0)
            pltpu.sync_copy(rows0, acc.at[ridx_all.at[j]], add=True)

        plsc.subcore_barrier()
        pltpu.sync_copy(acc.at[pl.ds(s * NPS, NPS)],
                        out_hbm.at[pl.ds(c * NPAD + s * NPS, NPS)])

    return body


@functools.lru_cache(maxsize=4)
def _get_sc_scatter(edge_base, epw_p, nch_p):
    mesh = plsc.VectorSubcoreMesh(core_axis_name="c", subcore_axis_name="s")
    return pl.kernel(
        _make_sc_body(edge_base, epw_p, nch_p),
        mesh=mesh,
        out_type=jax.ShapeDtypeStruct((NC * NPAD, D), jnp.float32),
        scratch_types=[
            pltpu.VMEM((epw_p,), jnp.int32),
            pltpu.VMEM((nch_p, CH), jnp.int32),
            pltpu.VMEM((CH, D), jnp.float32),
            pltpu.VMEM((CH, D), jnp.float32),
            pltpu.VMEM((CH, D), jnp.float32),
            pltpu.VMEM((CH, D), jnp.float32),
            pltpu.VMEM_SHARED((NPAD, D), jnp.float32),
            pltpu.SemaphoreType.DMA,
            pltpu.SemaphoreType.DMA,
            pltpu.SemaphoreType.DMA,
            pltpu.SemaphoreType.DMA,
        ],
    )


def _out_body(a0_ref, a1_ref, b0_ref, b1_ref, w_ref, o_ref):
    p = (a0_ref[0] + a1_ref[0]) + (b0_ref[0] + b1_ref[0])
    o_ref[...] = jnp.dot(p, w_ref[...], preferred_element_type=jnp.float32)


def _out_call(pa, pb, w_out):
    grid = (N // NB_OUT,)
    specs = [
        pl.BlockSpec((1, NB_OUT, D), lambda i: (0, i, 0)),
        pl.BlockSpec((1, NB_OUT, D), lambda i: (1, i, 0)),
    ]
    return pl.pallas_call(
        _out_body,
        grid=grid,
        in_specs=specs + specs + [pl.BlockSpec((D, D), lambda i: (0, 0))],
        out_specs=pl.BlockSpec((NB_OUT, D), lambda i: (i, 0)),
        out_shape=jax.ShapeDtypeStruct((N, D), jnp.float32),
    )(pa, pa, pb, pb, w_out)


def kernel(node_attrs, node_feats, edge_attrs, edge_feats, senders, receivers,
           W_sc, W1, W_m0, W_m1, W_m2, W_m3, W_out):
    wsc_r = W_sc.reshape(D, DA * D)
    w3f = (W_m3.reshape(H, D, DS).transpose(2, 0, 1).reshape(DS * H, D)
           * INV_SQRT_AVG)
    zeros = jnp.zeros((NPS, D), jnp.float32)
    rcv_a = receivers[:E_SPLIT].reshape(NW, NCH_A, CH)
    rcv_b = receivers[E_SPLIT:].reshape(NW, NCH_B, CH)

    nf = _nf_call(node_feats, W1)
    eft, eat = edge_feats.T, edge_attrs.T
    g_a = _edge_call(eft, eat, W_m0, W_m1, W_m2, w3f, 0, E_SPLIT)
    p_a = _get_sc_scatter(0, EPW_A, NCH_A)(nf, g_a, senders, rcv_a, zeros)
    w0_b = W_m0 + g_a[0, 0] * 0.0
    g_b = _edge_call(eft, eat, w0_b, W_m1, W_m2, w3f, E_SPLIT, E)
    p_b = _get_sc_scatter(E_SPLIT, EPW_B, NCH_B)(nf, g_b, senders, rcv_b, zeros)
    wsc_dep = wsc_r + g_b[0, 0] * 0.0
    sc = _sc_node_call(node_feats, node_attrs, wsc_dep)
    message = _out_call(p_a.reshape(NC, NPAD, D), p_b.reshape(NC, NPAD, D), W_out)
    return (message, sc)
